# Initial kernel scaffold; baseline (speedup 1.0000x reference)
import jax
import jax.numpy as jnp
from jax import lax
from jax.experimental import pallas as pl
from jax.experimental.pallas import tpu as pltpu
from jax.experimental.pallas import tpu_sc as plsc

N_NODES_K = 10000
N_EDGES_K = 320000
D_K = 128

NPAD = 10240
CHUNK = 128
N_CHUNKS = N_EDGES_K // CHUNK
NC, NS = 2, 16
NW = NC * NS
BASE_CHUNKS = N_CHUNKS // NW
EXTRA = N_CHUNKS - BASE_CHUNKS * NW
DEGW = 16
ROWS_PER_SUB = NPAD // NS


def _sc_scatter(src_hbm, dst_hbm, feat_hbm, acc_out, deg_out,
                acc_sh, deg_sh,
                src_idx, dst_idx, rows, ones_v, zrows, zdeg, sem):
    cid = lax.axis_index("c")
    sid = lax.axis_index("s")
    wid = sid * NC + cid

    def fill_ones(i, _):
        ones_v[i, :] = jnp.ones((16,), jnp.float32)
        return 0
    lax.fori_loop(0, CHUNK, fill_ones, 0)

    def fill_zrows(i, _):
        zrows[i // 8, pl.ds((i % 8) * 16, 16)] = jnp.zeros((16,), jnp.float32)
        return 0
    lax.fori_loop(0, 64 * (D_K // 16), fill_zrows, 0)

    def fill_zdeg(i, _):
        zdeg[i, :] = jnp.zeros((16,), jnp.float32)
        return 0
    lax.fori_loop(0, 64, fill_zdeg, 0)

    zbase = sid * ROWS_PER_SUB
    def zero_acc(k, _):
        pltpu.sync_copy(zrows, acc_sh.at[pl.ds(zbase + k * 64, 64)])
        pltpu.sync_copy(zdeg, deg_sh.at[pl.ds(zbase + k * 64, 64)])
        return 0
    lax.fori_loop(0, ROWS_PER_SUB // 64, zero_acc, 0)
    plsc.subcore_barrier()

    n_w = BASE_CHUNKS + jnp.where(wid < EXTRA, 1, 0)
    start_w = BASE_CHUNKS * wid + jnp.minimum(wid, EXTRA)

    def chunk_body(j, _):
        off = (start_w + j) * CHUNK
        pltpu.sync_copy(src_hbm.at[pl.ds(off, CHUNK)], src_idx)
        pltpu.sync_copy(dst_hbm.at[pl.ds(off, CHUNK)], dst_idx)
        pltpu.async_copy(feat_hbm.at[src_idx], rows, sem).wait()
        pltpu.sync_copy(rows, acc_sh.at[dst_idx], add=True)
        pltpu.sync_copy(ones_v, deg_sh.at[dst_idx], add=True)
        return 0
    lax.fori_loop(0, n_w, chunk_body, 0)
    plsc.subcore_barrier()

    obase = sid * ROWS_PER_SUB
    pltpu.sync_copy(acc_sh.at[pl.ds(obase, ROWS_PER_SUB)],
                    acc_out.at[cid, pl.ds(obase, ROWS_PER_SUB)])
    pltpu.sync_copy(deg_sh.at[pl.ds(obase, ROWS_PER_SUB)],
                    deg_out.at[cid, pl.ds(obase, ROWS_PER_SUB)])


def _make_sc_kernel():
    mesh = plsc.VectorSubcoreMesh(core_axis_name="c", subcore_axis_name="s")
    return pl.kernel(
        _sc_scatter,
        out_type=(
            jax.ShapeDtypeStruct((NC, NPAD, D_K), jnp.float32),
            jax.ShapeDtypeStruct((NC, NPAD, DEGW), jnp.float32),
        ),
        mesh=mesh,
        scratch_types=[
            pltpu.VMEM_SHARED((NPAD, D_K), jnp.float32),
            pltpu.VMEM_SHARED((NPAD, DEGW), jnp.float32),
            pltpu.VMEM((CHUNK,), jnp.int32),
            pltpu.VMEM((CHUNK,), jnp.int32),
            pltpu.VMEM((CHUNK, D_K), jnp.float32),
            pltpu.VMEM((CHUNK, DEGW), jnp.float32),
            pltpu.VMEM((64, D_K), jnp.float32),
            pltpu.VMEM((64, DEGW), jnp.float32),
            pltpu.SemaphoreType.DMA,
        ],
    )


def _tc_finish(acc_ref, deg_ref, w_ref, b_ref, out_ref):
    a = acc_ref[0] + acc_ref[1]
    d = deg_ref[0, :, :1] + deg_ref[1, :, :1]
    h = jnp.where(d > 0, a / jnp.maximum(d, 1.0), 0.0)
    y = lax.dot_general(h, w_ref[...], (((1,), (1,)), ((), ())),
                        preferred_element_type=jnp.float32)
    out_ref[...] = y + b_ref[...]


def kernel(feature, edge_index, W, b):
    src = edge_index[0]
    dst = edge_index[1]

    acc, deg = _make_sc_kernel()(src, dst, feature)

    R = 1000
    grid = (N_NODES_K // R,)
    out = pl.pallas_call(
        _tc_finish,
        grid=grid,
        in_specs=[
            pl.BlockSpec((NC, R, D_K), lambda i: (0, i, 0)),
            pl.BlockSpec((NC, R, DEGW), lambda i: (0, i, 0)),
            pl.BlockSpec((D_K, D_K), lambda i: (0, 0)),
            pl.BlockSpec((1, D_K), lambda i: (0, 0)),
        ],
        out_specs=pl.BlockSpec((R, D_K), lambda i: (i, 0)),
        out_shape=jax.ShapeDtypeStruct((N_NODES_K, D_K), jnp.float32),
    )(acc, deg, W, b.reshape(1, D_K))
    return out


# speedup vs baseline: 4.8207x; 4.8207x over previous
import jax
import jax.numpy as jnp
from jax import lax
from jax.experimental import pallas as pl
from jax.experimental.pallas import tpu as pltpu
from jax.experimental.pallas import tpu_sc as plsc

N_NODES_K = 10000
N_EDGES_K = 320000
D_K = 128

NPAD = 10240
CHUNK = 80
N_CHUNKS = N_EDGES_K // CHUNK
NC, NS = 2, 16
NW = NC * NS
CHUNKS_PER_TILE = N_CHUNKS // NW
ROWS_PER_SUB = NPAD // NS


def _fill(ref, n_rows, value):
    def body(i, _):
        ref[i // 8, pl.ds((i % 8) * 16, 16)] = jnp.full((16,), value,
                                                        jnp.float32)
        return 0
    lax.fori_loop(0, n_rows * (D_K // 16), body, 0)


def _sc_scatter(src_hbm, dst_hbm, feat_hbm, acc_out, deg_out,
                acc_sh, src_idx, dst_idx, rows, sem):
    cid = lax.axis_index("c")
    sid = lax.axis_index("s")
    wid = sid * NC + cid
    zbase = sid * ROWS_PER_SUB

    def zero_acc():
        _fill(rows, CHUNK, 0.0)
        def z(k, _):
            pltpu.sync_copy(rows, acc_sh.at[pl.ds(zbase + k * CHUNK, CHUNK)])
            return 0
        lax.fori_loop(0, ROWS_PER_SUB // CHUNK, z, 0)

    def copy_out(dst_3d):
        def c(k, _):
            o = zbase + k * CHUNK
            pltpu.sync_copy(acc_sh.at[pl.ds(o, CHUNK)], rows)
            pltpu.sync_copy(rows, dst_3d.at[cid, pl.ds(o, CHUNK)])
            return 0
        lax.fori_loop(0, ROWS_PER_SUB // CHUNK, c, 0)

    zero_acc()
    plsc.subcore_barrier()

    def chunk1(j, _):
        off = wid * (CHUNKS_PER_TILE * CHUNK) + j * CHUNK
        pltpu.sync_copy(src_hbm.at[pl.ds(off, CHUNK)], src_idx)
        pltpu.sync_copy(dst_hbm.at[pl.ds(off, CHUNK)], dst_idx)
        pltpu.async_copy(feat_hbm.at[src_idx], rows, sem).wait()
        pltpu.sync_copy(rows, acc_sh.at[dst_idx], add=True)
        return 0
    lax.fori_loop(0, CHUNKS_PER_TILE, chunk1, 0)
    plsc.subcore_barrier()

    copy_out(acc_out)
    zero_acc()
    plsc.subcore_barrier()

    _fill(rows, CHUNK, 1.0)
    def chunk2(j, _):
        off = wid * (CHUNKS_PER_TILE * CHUNK) + j * CHUNK
        pltpu.sync_copy(dst_hbm.at[pl.ds(off, CHUNK)], dst_idx)
        pltpu.sync_copy(rows, acc_sh.at[dst_idx], add=True)
        return 0
    lax.fori_loop(0, CHUNKS_PER_TILE, chunk2, 0)
    plsc.subcore_barrier()

    copy_out(deg_out)


def _make_sc_kernel():
    mesh = plsc.VectorSubcoreMesh(core_axis_name="c", subcore_axis_name="s")
    return pl.kernel(
        _sc_scatter,
        out_type=(
            jax.ShapeDtypeStruct((NC, NPAD, D_K), jnp.float32),
            jax.ShapeDtypeStruct((NC, NPAD, D_K), jnp.float32),
        ),
        mesh=mesh,
        scratch_types=[
            pltpu.VMEM_SHARED((NPAD, D_K), jnp.float32),
            pltpu.VMEM((CHUNK,), jnp.int32),
            pltpu.VMEM((CHUNK,), jnp.int32),
            pltpu.VMEM((CHUNK, D_K), jnp.float32),
            pltpu.SemaphoreType.DMA,
        ],
    )


def _tc_finish(acc_ref, deg_ref, w_ref, b_ref, out_ref):
    a = acc_ref[0] + acc_ref[1]
    d = deg_ref[0, :, :1] + deg_ref[1, :, :1]
    h = jnp.where(d > 0, a / jnp.maximum(d, 1.0), 0.0)
    y = lax.dot_general(h, w_ref[...], (((1,), (1,)), ((), ())),
                        preferred_element_type=jnp.float32)
    out_ref[...] = y + b_ref[...]


def kernel(feature, edge_index, W, b):
    src = edge_index[0]
    dst = edge_index[1]

    acc, deg = _make_sc_kernel()(src, dst, feature)

    R = 1280
    grid = (NPAD // R,)
    out = pl.pallas_call(
        _tc_finish,
        grid=grid,
        in_specs=[
            pl.BlockSpec((NC, R, D_K), lambda i: (0, i, 0)),
            pl.BlockSpec((NC, R, D_K), lambda i: (0, i, 0)),
            pl.BlockSpec((D_K, D_K), lambda i: (0, 0)),
            pl.BlockSpec((1, D_K), lambda i: (0, 0)),
        ],
        out_specs=pl.BlockSpec((R, D_K), lambda i: (i, 0)),
        out_shape=jax.ShapeDtypeStruct((N_NODES_K, D_K), jnp.float32),
    )(acc, deg, W, b.reshape(1, D_K))
    return out


# speedup vs baseline: 9.3485x; 1.9392x over previous
import jax
import jax.numpy as jnp
from jax import lax
from jax.experimental import pallas as pl
from jax.experimental.pallas import tpu as pltpu
from jax.experimental.pallas import tpu_sc as plsc

N_NODES_K = 10000
N_EDGES_K = 320000
D_K = 128

NPAD = 10240
CHUNK = 80
N_CHUNKS = N_EDGES_K // CHUNK
NC, NS = 2, 16
NW = NC * NS
CHUNKS_PER_TILE = N_CHUNKS // NW
IB = 25
NB = CHUNKS_PER_TILE // IB
ROWS_PER_SUB = NPAD // NS


def _fill(ref2d, value):
    n = ref2d.shape[0] * ref2d.shape[1] // 16
    def body(i, _):
        ref2d[i // 8, pl.ds((i % 8) * 16, 16)] = jnp.full((16,), value,
                                                          jnp.float32)
        return 0
    lax.fori_loop(0, n, body, 0)


def _sc_scatter(src_hbm, dst_hbm, feat_hbm, acc_out, deg_out,
                acc_sh, sidx, didx, rows, sem, sem2):
    cid = lax.axis_index("c")
    sid = lax.axis_index("s")
    wid = sid * NC + cid
    zbase = sid * ROWS_PER_SUB

    def zero_acc():
        _fill(rows.at[0], 0.0)
        def z(k, _):
            pltpu.sync_copy(rows.at[0],
                            acc_sh.at[pl.ds(zbase + k * CHUNK, CHUNK)])
            return 0
        lax.fori_loop(0, ROWS_PER_SUB // CHUNK, z, 0)

    def copy_out(dst_3d, stage):
        def c(k, _):
            o = zbase + k * CHUNK
            pltpu.sync_copy(acc_sh.at[pl.ds(o, CHUNK)], stage)
            pltpu.sync_copy(stage, dst_3d.at[cid, pl.ds(o, CHUNK)])
            return 0
        lax.fori_loop(0, ROWS_PER_SUB // CHUNK, c, 0)

    zero_acc()
    plsc.subcore_barrier()

    def blk1(bi, _):
        bid = wid * NB + bi
        pltpu.sync_copy(src_hbm.at[bid], sidx)
        pltpu.sync_copy(dst_hbm.at[bid], didx)
        pltpu.async_copy(feat_hbm.at[sidx.at[0]], rows.at[0], sem)
        def chunk1(j, _):
            p = lax.rem(j, 2)
            @pl.when(j + 1 < IB)
            def _():
                pltpu.async_copy(feat_hbm.at[sidx.at[j + 1]],
                                 rows.at[1 - p], sem)
            pltpu.make_async_copy(feat_hbm.at[sidx.at[j]],
                                  rows.at[p], sem).wait()
            pltpu.sync_copy(rows.at[p], acc_sh.at[didx.at[j]], add=True)
            return 0
        lax.fori_loop(0, IB, chunk1, 0)
        return 0
    lax.fori_loop(0, NB, blk1, 0)
    plsc.subcore_barrier()

    copy_out(acc_out, rows.at[0])
    zero_acc()
    plsc.subcore_barrier()

    _fill(rows.at[0], 1.0)
    def blk2(bi, _):
        bid = wid * NB + bi
        pltpu.sync_copy(dst_hbm.at[bid], didx)
        def fire(j, _):
            pltpu.async_copy(rows.at[0], acc_sh.at[didx.at[j]], sem2,
                             add=True)
            return 0
        lax.fori_loop(0, IB, fire, 0)
        def drain(j, _):
            pltpu.make_async_copy(rows.at[0], acc_sh.at[didx.at[j]],
                                  sem2).wait()
            return 0
        lax.fori_loop(0, IB, drain, 0)
        return 0
    lax.fori_loop(0, NB, blk2, 0)
    plsc.subcore_barrier()

    copy_out(deg_out, rows.at[1])


def _make_sc_kernel():
    mesh = plsc.VectorSubcoreMesh(core_axis_name="c", subcore_axis_name="s")
    return pl.kernel(
        _sc_scatter,
        out_type=(
            jax.ShapeDtypeStruct((NC, NPAD, D_K), jnp.float32),
            jax.ShapeDtypeStruct((NC, NPAD, D_K), jnp.float32),
        ),
        mesh=mesh,
        scratch_types=[
            pltpu.VMEM_SHARED((NPAD, D_K), jnp.float32),
            pltpu.VMEM((IB, CHUNK), jnp.int32),
            pltpu.VMEM((IB, CHUNK), jnp.int32),
            pltpu.VMEM((2, CHUNK, D_K), jnp.float32),
            pltpu.SemaphoreType.DMA,
            pltpu.SemaphoreType.DMA,
        ],
    )


def _tc_finish(acc_ref, deg_ref, w_ref, b_ref, out_ref):
    a = acc_ref[0] + acc_ref[1]
    d = deg_ref[0, :, :1] + deg_ref[1, :, :1]
    h = jnp.where(d > 0, a / jnp.maximum(d, 1.0), 0.0)
    y = lax.dot_general(h, w_ref[...], (((1,), (1,)), ((), ())),
                        preferred_element_type=jnp.float32)
    out_ref[...] = y + b_ref[...]


def kernel(feature, edge_index, W, b):
    src3d = edge_index[0].reshape(NW * NB, IB, CHUNK)
    dst3d = edge_index[1].reshape(NW * NB, IB, CHUNK)

    acc, deg = _make_sc_kernel()(src3d, dst3d, feature)

    R = 1280
    grid = (NPAD // R,)
    out = pl.pallas_call(
        _tc_finish,
        grid=grid,
        in_specs=[
            pl.BlockSpec((NC, R, D_K), lambda i: (0, i, 0)),
            pl.BlockSpec((NC, R, D_K), lambda i: (0, i, 0)),
            pl.BlockSpec((D_K, D_K), lambda i: (0, 0)),
            pl.BlockSpec((1, D_K), lambda i: (0, 0)),
        ],
        out_specs=pl.BlockSpec((R, D_K), lambda i: (i, 0)),
        out_shape=jax.ShapeDtypeStruct((N_NODES_K, D_K), jnp.float32),
    )(acc, deg, W, b.reshape(1, D_K))
    return out


# speedup vs baseline: 9.5914x; 1.0260x over previous
import jax
import jax.numpy as jnp
from jax import lax
from jax.experimental import pallas as pl
from jax.experimental.pallas import tpu as pltpu
from jax.experimental.pallas import tpu_sc as plsc

N_NODES_K = 10000
N_EDGES_K = 320000
D_K = 128

NPAD = 10240
CHUNK = 80
N_CHUNKS = N_EDGES_K // CHUNK
NC, NS = 2, 16
NW = NC * NS
CHUNKS_PER_TILE = N_CHUNKS // NW
IB = 25
NB = CHUNKS_PER_TILE // IB
ROWS_PER_SUB = NPAD // NS


def _fill(ref2d, value):
    n = ref2d.shape[0] * ref2d.shape[1] // 16
    def body(i, _):
        ref2d[i // 8, pl.ds((i % 8) * 16, 16)] = jnp.full((16,), value,
                                                          jnp.float32)
        return 0
    lax.fori_loop(0, n, body, 0)


def _sc_scatter(src_hbm, dst_hbm, feat_hbm, acc_out, deg_out,
                acc_sh, sidx, didx, rows, sem, sem2):
    cid = lax.axis_index("c")
    sid = lax.axis_index("s")
    wid = sid * NC + cid
    zbase = sid * ROWS_PER_SUB

    def zero_acc():
        _fill(rows.at[0], 0.0)
        def z(k, _):
            pltpu.sync_copy(rows.at[0],
                            acc_sh.at[pl.ds(zbase + k * CHUNK, CHUNK)])
            return 0
        lax.fori_loop(0, ROWS_PER_SUB // CHUNK, z, 0)

    def copy_out(dst_3d, stage):
        def c(k, _):
            o = zbase + k * CHUNK
            pltpu.sync_copy(acc_sh.at[pl.ds(o, CHUNK)], stage)
            pltpu.sync_copy(stage, dst_3d.at[cid, pl.ds(o, CHUNK)])
            return 0
        lax.fori_loop(0, ROWS_PER_SUB // CHUNK, c, 0)

    zero_acc()
    plsc.subcore_barrier()

    def blk1(bi, _):
        bid = wid * NB + bi
        pltpu.sync_copy(src_hbm.at[bid], sidx)
        pltpu.sync_copy(dst_hbm.at[bid], didx)
        pltpu.async_copy(feat_hbm.at[sidx.at[0]], rows.at[0], sem)
        def chunk1(j, _):
            p = lax.rem(j, 2)
            @pl.when(j + 1 < IB)
            def _():
                pltpu.async_copy(feat_hbm.at[sidx.at[j + 1]],
                                 rows.at[1 - p], sem)
            pltpu.make_async_copy(feat_hbm.at[sidx.at[j]],
                                  rows.at[p], sem).wait()
            pltpu.sync_copy(rows.at[p], acc_sh.at[didx.at[j]], add=True)
            return 0
        lax.fori_loop(0, IB, chunk1, 0)
        return 0
    lax.fori_loop(0, NB, blk1, 0)
    plsc.subcore_barrier()

    copy_out(acc_out, rows.at[0])
    plsc.subcore_barrier()

    _fill(rows.at[0], 1.0)
    def blk2(bi, _):
        bid = wid * NB + bi
        pltpu.sync_copy(dst_hbm.at[bid], didx)
        def fire(j, _):
            pltpu.async_copy(rows.at[0], acc_sh.at[didx.at[j]], sem2,
                             add=True)
            return 0
        lax.fori_loop(0, IB, fire, 0)
        def drain(j, _):
            pltpu.make_async_copy(rows.at[0], acc_sh.at[didx.at[j]],
                                  sem2).wait()
            return 0
        lax.fori_loop(0, IB, drain, 0)
        return 0
    lax.fori_loop(0, NB, blk2, 0)
    plsc.subcore_barrier()

    copy_out(deg_out, rows.at[1])


def _make_sc_kernel():
    mesh = plsc.VectorSubcoreMesh(core_axis_name="c", subcore_axis_name="s")
    return pl.kernel(
        _sc_scatter,
        out_type=(
            jax.ShapeDtypeStruct((NC, NPAD, D_K), jnp.float32),
            jax.ShapeDtypeStruct((NC, NPAD, D_K), jnp.float32),
        ),
        mesh=mesh,
        scratch_types=[
            pltpu.VMEM_SHARED((NPAD, D_K), jnp.float32),
            pltpu.VMEM((IB, CHUNK), jnp.int32),
            pltpu.VMEM((IB, CHUNK), jnp.int32),
            pltpu.VMEM((2, CHUNK, D_K), jnp.float32),
            pltpu.SemaphoreType.DMA,
            pltpu.SemaphoreType.DMA,
        ],
    )


def _tc_finish(acc_ref, deg_ref, w_ref, b_ref, out_ref):
    a = acc_ref[0] + acc_ref[1]
    d = deg_ref[0, :, :1] + deg_ref[1, :, :1] - a[:, :1]
    d = jnp.round(d)
    h = jnp.where(d > 0, a / jnp.maximum(d, 1.0), 0.0)
    y = lax.dot_general(h, w_ref[...], (((1,), (1,)), ((), ())),
                        preferred_element_type=jnp.float32)
    out_ref[...] = y + b_ref[...]


def kernel(feature, edge_index, W, b):
    src3d = edge_index[0].reshape(NW * NB, IB, CHUNK)
    dst3d = edge_index[1].reshape(NW * NB, IB, CHUNK)

    acc, deg = _make_sc_kernel()(src3d, dst3d, feature)

    R = 1280
    grid = (NPAD // R,)
    out = pl.pallas_call(
        _tc_finish,
        grid=grid,
        in_specs=[
            pl.BlockSpec((NC, R, D_K), lambda i: (0, i, 0)),
            pl.BlockSpec((NC, R, D_K), lambda i: (0, i, 0)),
            pl.BlockSpec((D_K, D_K), lambda i: (0, 0)),
            pl.BlockSpec((1, D_K), lambda i: (0, 0)),
        ],
        out_specs=pl.BlockSpec((R, D_K), lambda i: (i, 0)),
        out_shape=jax.ShapeDtypeStruct((N_NODES_K, D_K), jnp.float32),
    )(acc, deg, W, b.reshape(1, D_K))
    return out


# speedup vs baseline: 9.8764x; 1.0297x over previous
import jax
import jax.numpy as jnp
from jax import lax
from jax.experimental import pallas as pl
from jax.experimental.pallas import tpu as pltpu
from jax.experimental.pallas import tpu_sc as plsc

N_NODES_K = 10000
N_EDGES_K = 320000
D_K = 128

NPAD = 10240
CHUNK = 80
N_CHUNKS = N_EDGES_K // CHUNK
NC, NS = 2, 16
NW = NC * NS
CHUNKS_PER_TILE = N_CHUNKS // NW
IB = 25
NB = CHUNKS_PER_TILE // IB
ROWS_PER_SUB = NPAD // NS
DEGW = 16


def _fill(ref2d, value):
    n = ref2d.shape[0] * ref2d.shape[1] // 16
    def body(i, _):
        ref2d[i // 8, pl.ds((i % 8) * 16, 16)] = jnp.full((16,), value,
                                                          jnp.float32)
        return 0
    lax.fori_loop(0, n, body, 0)


def _sc_scatter(src_hbm, dst_hbm, feat_hbm, acc_out, deg_out,
                acc_sh, sidx, didx, rows, semA, semB, semI):
    cid = lax.axis_index("c")
    sid = lax.axis_index("s")
    wid = sid * NC + cid
    zbase = sid * ROWS_PER_SUB
    blk0 = wid * NB

    def zero_acc():
        _fill(rows.at[0], 0.0)
        def z(k, _):
            pltpu.sync_copy(rows.at[0],
                            acc_sh.at[pl.ds(zbase + k * CHUNK, CHUNK)])
            return 0
        lax.fori_loop(0, ROWS_PER_SUB // CHUNK, z, 0)

    zero_acc()
    plsc.subcore_barrier()

    pltpu.sync_copy(src_hbm.at[blk0], sidx.at[0])
    pltpu.sync_copy(dst_hbm.at[blk0], didx.at[0])

    def blk1(bi, _):
        s = lax.rem(bi, 2)
        nxt = 1 - s
        @pl.when(bi + 1 < NB)
        def _():
            pltpu.async_copy(src_hbm.at[blk0 + bi + 1], sidx.at[nxt], semI)
            pltpu.async_copy(dst_hbm.at[blk0 + bi + 1], didx.at[nxt], semI)
        pltpu.async_copy(feat_hbm.at[sidx.at[s, 0]], rows.at[0], semA)
        def chunk1(j, _):
            p = lax.rem(j, 2)
            @pl.when(j + 1 < IB)
            def _():
                pltpu.async_copy(feat_hbm.at[sidx.at[s, j + 1]],
                                 rows.at[1 - p], semA)
            pltpu.make_async_copy(feat_hbm.at[sidx.at[s, j]],
                                  rows.at[p], semA).wait()
            pltpu.sync_copy(rows.at[p], acc_sh.at[didx.at[s, j]], add=True)
            return 0
        lax.fori_loop(0, IB, chunk1, 0)
        @pl.when(bi + 1 < NB)
        def _():
            pltpu.make_async_copy(src_hbm.at[blk0 + bi + 1], sidx.at[nxt],
                                  semI).wait()
            pltpu.make_async_copy(dst_hbm.at[blk0 + bi + 1], didx.at[nxt],
                                  semI).wait()
        return 0
    lax.fori_loop(0, NB, blk1, 0)
    plsc.subcore_barrier()

    def acc_copy(k, _):
        o = zbase + k * CHUNK
        pltpu.sync_copy(acc_sh.at[pl.ds(o, CHUNK)], rows.at[0])
        pltpu.sync_copy(rows.at[0], acc_out.at[cid, pl.ds(o, CHUNK)])
        return 0
    lax.fori_loop(0, ROWS_PER_SUB // CHUNK, acc_copy, 0)
    plsc.subcore_barrier()

    _fill(rows.at[0], 1.0)
    pltpu.sync_copy(dst_hbm.at[blk0], didx.at[0])

    def fire_blk(s):
        def fire(j, _):
            pltpu.async_copy(rows.at[0], acc_sh.at[didx.at[s, j]], semB,
                             add=True)
            return 0
        lax.fori_loop(0, IB, fire, 0)

    def drain_blk(s):
        def drain(j, _):
            pltpu.make_async_copy(rows.at[0], acc_sh.at[didx.at[s, j]],
                                  semB).wait()
            return 0
        lax.fori_loop(0, IB, drain, 0)

    fire_blk(0)
    def blk2(bi, _):
        s = lax.rem(bi, 2)
        pltpu.sync_copy(dst_hbm.at[blk0 + bi], didx.at[s])
        fire_blk(s)
        drain_blk(1 - s)
        return 0
    lax.fori_loop(1, NB, blk2, 0)
    drain_blk((NB - 1) % 2)
    plsc.subcore_barrier()

    def deg_copy(k, _):
        o = zbase + k * CHUNK
        pltpu.sync_copy(acc_sh.at[pl.ds(o, CHUNK)], rows.at[0])
        pltpu.sync_copy(rows.at[0], deg_out.at[cid, pl.ds(o, CHUNK)])
        return 0
    lax.fori_loop(0, ROWS_PER_SUB // CHUNK, deg_copy, 0)


def _make_sc_kernel():
    mesh = plsc.VectorSubcoreMesh(core_axis_name="c", subcore_axis_name="s")
    return pl.kernel(
        _sc_scatter,
        out_type=(
            jax.ShapeDtypeStruct((NC, NPAD, D_K), jnp.float32),
            jax.ShapeDtypeStruct((NC, NPAD, D_K), jnp.float32),
        ),
        mesh=mesh,
        scratch_types=[
            pltpu.VMEM_SHARED((NPAD, D_K), jnp.float32),
            pltpu.VMEM((2, IB, CHUNK), jnp.int32),
            pltpu.VMEM((2, IB, CHUNK), jnp.int32),
            pltpu.VMEM((2, CHUNK, D_K), jnp.float32),
            pltpu.SemaphoreType.DMA,
            pltpu.SemaphoreType.DMA,
            pltpu.SemaphoreType.DMA,
        ],
    )


def _tc_finish(acc_ref, deg_ref, w_ref, b_ref, out_ref):
    a = acc_ref[0] + acc_ref[1]
    d = deg_ref[0, :, :1] + deg_ref[1, :, :1] - a[:, :1]
    d = jnp.round(d)
    h = jnp.where(d > 0, a / jnp.maximum(d, 1.0), 0.0)
    y = lax.dot_general(h, w_ref[...], (((1,), (1,)), ((), ())),
                        preferred_element_type=jnp.float32)
    out_ref[...] = y + b_ref[...]


def kernel(feature, edge_index, W, b):
    src3d = edge_index[0].reshape(NW * NB, IB, CHUNK)
    dst3d = edge_index[1].reshape(NW * NB, IB, CHUNK)

    acc, deg = _make_sc_kernel()(src3d, dst3d, feature)

    R = 1280
    grid = (NPAD // R,)
    out = pl.pallas_call(
        _tc_finish,
        grid=grid,
        in_specs=[
            pl.BlockSpec((NC, R, D_K), lambda i: (0, i, 0)),
            pl.BlockSpec((NC, R, D_K), lambda i: (0, i, 0)),
            pl.BlockSpec((D_K, D_K), lambda i: (0, 0)),
            pl.BlockSpec((1, D_K), lambda i: (0, 0)),
        ],
        out_specs=pl.BlockSpec((R, D_K), lambda i: (i, 0)),
        out_shape=jax.ShapeDtypeStruct((N_NODES_K, D_K), jnp.float32),
    )(acc, deg, W, b.reshape(1, D_K))
    return out
